# baseline (speedup 1.0000x reference)
import jax
import jax.numpy as jnp
from jax import lax
from jax.experimental import pallas as pl
from jax.experimental.pallas import tpu as pltpu
from jax.experimental.pallas import tpu_sc as plsc

MODEL_DIM = 64
SCALE = 8.0

NC = 2
NS = 16
LANE = 16
IDX_W = 128

S0 = 4096
S1 = 200
CBLK = S0 // IDX_W
W_S1 = 8
W_C = 4
S1_PER_W = S1 // W_S1
CG_PER_W = CBLK // W_C
UNITS_PER_W = S1_PER_W * CG_PER_W


def _body(idx_hbm, t128_hbm, out_hbm, idx_v, idx2_v, rows2_v, outT_v, gsem):
    wid = lax.axis_index("s") * NC + lax.axis_index("c")
    s1_base = (wid // W_C) * S1_PER_W
    c_base = (wid % W_C) * CG_PER_W
    lanes = lax.iota(jnp.int32, LANE)

    @pl.loop(0, UNITS_PER_W)
    def _unit(b):
        s1 = s1_base + b // CG_PER_W
        c = c_base + lax.rem(b, CG_PER_W)
        pltpu.sync_copy(idx_hbm.at[s1, pl.ds(c * IDX_W, IDX_W)], idx_v)

        for k in range(IDX_W // LANE):
            sl = pl.ds(k * LANE, LANE)
            idx2_v[sl] = lax.shift_right_logical(idx_v[sl], 1)

        pltpu.async_copy(t128_hbm.at[idx2_v], rows2_v, gsem).wait()

        for g in range(IDX_W // LANE):
            sl = pl.ds(g * LANE, LANE)
            rows_g = lanes + (g * LANE)
            col_base = (idx_v[sl] & 1) * MODEL_DIM

            @pl.loop(0, MODEL_DIM)
            def _d(d, _rows_g=rows_g, _col=col_base, _sl=sl):
                vals = plsc.load_gather(rows2_v, [_rows_g, _col + d])
                outT_v[d >> 3, d & 7, _sl] = vals * SCALE

        pltpu.sync_copy(outT_v, out_hbm.at[s1, :, c])


def kernel(x, table):
    xt = x.T.astype(jnp.int32)
    t128 = table.reshape(table.shape[0] // 2, 2 * MODEL_DIM)

    run = pl.kernel(
        _body,
        out_type=jax.ShapeDtypeStruct(
            (S1, MODEL_DIM // 8, CBLK, 8, IDX_W), jnp.float32),
        mesh=plsc.VectorSubcoreMesh(core_axis_name="c", subcore_axis_name="s"),
        scratch_types=[
            pltpu.VMEM((IDX_W,), jnp.int32),
            pltpu.VMEM((IDX_W,), jnp.int32),
            pltpu.VMEM((IDX_W, IDX_W), jnp.float32),
            pltpu.VMEM((MODEL_DIM // 8, 8, IDX_W), jnp.float32),
            pltpu.SemaphoreType.DMA,
        ],
        compiler_params=pltpu.CompilerParams(
            use_tc_tiling_on_sc=False, needs_layout_passes=False),
    )
    out5 = run(xt, t128)
    return out5.transpose(2, 4, 0, 1, 3).reshape(S0, S1, MODEL_DIM)


# speedup vs baseline: 1.0511x; 1.0511x over previous
import jax
import jax.numpy as jnp
from jax import lax
from jax.experimental import pallas as pl
from jax.experimental.pallas import tpu as pltpu
from jax.experimental.pallas import tpu_sc as plsc

MODEL_DIM = 64
SCALE = 8.0

NC = 2
NS = 16
LANE = 16
IDX_W = 128
PADW = 129

S0 = 4096
S1 = 200
CBLK = S0 // IDX_W
W_S1 = 8
W_C = 4
S1_PER_W = S1 // W_S1
CG_PER_W = CBLK // W_C
UNITS_PER_W = S1_PER_W * CG_PER_W
NG = IDX_W // LANE


def _body(idx_hbm, t128_hbm, out_hbm, idx_v, idx2_v, rows2_v, rowsP_v,
          outT_v, gsem):
    wid = lax.axis_index("s") * NC + lax.axis_index("c")
    s1_base = (wid // W_C) * S1_PER_W
    c_base = (wid % W_C) * CG_PER_W
    lanes = lax.iota(jnp.int32, LANE)

    @pl.loop(0, UNITS_PER_W)
    def _unit(b):
        s1 = s1_base + b // CG_PER_W
        c = c_base + lax.rem(b, CG_PER_W)
        pltpu.sync_copy(idx_hbm.at[s1, pl.ds(c * IDX_W, IDX_W)], idx_v)

        for k in range(NG):
            sl = pl.ds(k * LANE, LANE)
            idx2_v[sl] = lax.shift_right_logical(idx_v[sl], 1)

        pltpu.async_copy(t128_hbm.at[idx2_v], rows2_v, gsem).wait()

        @pl.loop(0, IDX_W)
        def _stage(j):
            for k in range(2 * MODEL_DIM // LANE):
                rowsP_v[pl.ds(j * PADW + k * LANE, LANE)] = (
                    rows2_v[j, pl.ds(k * LANE, LANE)])

        base_vecs = [
            (lanes + g * LANE) * PADW
            + (idx_v[pl.ds(g * LANE, LANE)] & 1) * MODEL_DIM
            for g in range(NG)
        ]

        @pl.loop(0, MODEL_DIM // 8)
        def _d8(d8):
            for dl in range(8):
                d = d8 * 8 + dl
                for g in range(NG):
                    vals = plsc.load_gather(rowsP_v, [base_vecs[g] + d])
                    outT_v[d8, dl, pl.ds(g * LANE, LANE)] = vals * SCALE

        pltpu.sync_copy(outT_v, out_hbm.at[s1, :, c])


def kernel(x, table):
    xt = x.T.astype(jnp.int32)
    t128 = table.reshape(table.shape[0] // 2, 2 * MODEL_DIM)

    run = pl.kernel(
        _body,
        out_type=jax.ShapeDtypeStruct(
            (S1, MODEL_DIM // 8, CBLK, 8, IDX_W), jnp.float32),
        mesh=plsc.VectorSubcoreMesh(core_axis_name="c", subcore_axis_name="s"),
        scratch_types=[
            pltpu.VMEM((IDX_W,), jnp.int32),
            pltpu.VMEM((IDX_W,), jnp.int32),
            pltpu.VMEM((IDX_W, 2 * MODEL_DIM), jnp.float32),
            pltpu.VMEM((IDX_W * PADW,), jnp.float32),
            pltpu.VMEM((MODEL_DIM // 8, 8, IDX_W), jnp.float32),
            pltpu.SemaphoreType.DMA,
        ],
        compiler_params=pltpu.CompilerParams(
            use_tc_tiling_on_sc=True, needs_layout_passes=False),
    )
    out5 = run(xt, t128)
    return out5.transpose(2, 4, 0, 1, 3).reshape(S0, S1, MODEL_DIM)


# speedup vs baseline: 1.0706x; 1.0186x over previous
import jax
import jax.numpy as jnp
from jax import lax
from jax.experimental import pallas as pl
from jax.experimental.pallas import tpu as pltpu
from jax.experimental.pallas import tpu_sc as plsc

MODEL_DIM = 64
SCALE = 8.0

NC = 2
NS = 16
LANE = 16
IDX_W = 128

S0 = 4096
S1 = 200
CBLK = S0 // IDX_W
W_S1 = 8
W_C = 4
S1_PER_W = S1 // W_S1
CG_PER_W = CBLK // W_C
UNITS_PER_W = S1_PER_W * CG_PER_W
NG = IDX_W // LANE


def _body(idx_hbm, t128_hbm, out_hbm, idx_v, idx2_v, rows2_v, sel_v, gsem):
    wid = lax.axis_index("s") * NC + lax.axis_index("c")
    s1_base = (wid // W_C) * S1_PER_W
    c_base = (wid % W_C) * CG_PER_W

    @pl.loop(0, UNITS_PER_W)
    def _unit(b):
        s1 = s1_base + b // CG_PER_W
        c = c_base + lax.rem(b, CG_PER_W)
        pltpu.sync_copy(idx_hbm.at[s1, pl.ds(c * IDX_W, IDX_W)],
                        idx_v.at[pl.ds(0, IDX_W)])

        for k in range(NG):
            sl = pl.ds(k * LANE, LANE)
            idx2_v[sl] = lax.shift_right_logical(idx_v[sl], 1)

        pltpu.async_copy(t128_hbm.at[idx2_v], rows2_v, gsem).wait()

        @pl.loop(0, IDX_W)
        def _sel(j):
            par = (idx_v[pl.ds(j, LANE)][0] & 1) * MODEL_DIM
            for k in range(MODEL_DIM // LANE):
                sel_v[j, pl.ds(k * LANE, LANE)] = (
                    rows2_v[j, pl.ds(par + k * LANE, LANE)] * SCALE)

        pltpu.sync_copy(sel_v, out_hbm.at[pl.ds(c * IDX_W, IDX_W), s1])


def kernel(x, table):
    xt = x.T.astype(jnp.int32)
    t128 = table.reshape(table.shape[0] // 2, 2 * MODEL_DIM)

    run = pl.kernel(
        _body,
        out_type=jax.ShapeDtypeStruct((S0, S1, MODEL_DIM), jnp.float32),
        mesh=plsc.VectorSubcoreMesh(core_axis_name="c", subcore_axis_name="s"),
        scratch_types=[
            pltpu.VMEM((IDX_W + LANE,), jnp.int32),
            pltpu.VMEM((IDX_W,), jnp.int32),
            pltpu.VMEM((IDX_W, 2 * MODEL_DIM), jnp.float32),
            pltpu.VMEM((IDX_W, MODEL_DIM), jnp.float32),
            pltpu.SemaphoreType.DMA,
        ],
        compiler_params=pltpu.CompilerParams(
            use_tc_tiling_on_sc=True, needs_layout_passes=False),
    )
    return run(xt, t128)


# speedup vs baseline: 1.8228x; 1.7026x over previous
import jax
import jax.numpy as jnp
from jax import lax
from jax.experimental import pallas as pl
from jax.experimental.pallas import tpu as pltpu
from jax.experimental.pallas import tpu_sc as plsc

MODEL_DIM = 64
SCALE = 8.0

NC = 2
NS = 16
LANE = 16
IDX_W = 128
UNITS_PER_BATCH = 4
BATCH_ROWS = UNITS_PER_BATCH * IDX_W

S0 = 4096
S1 = 200
CBLK = S0 // IDX_W
W_S1 = 8
W_C = 4
S1_PER_W = S1 // W_S1
CG_PER_W = CBLK // W_C
NBATCH = S1_PER_W * (CG_PER_W // UNITS_PER_BATCH)


def _body(idx_hbm, table_hbm, out_hbm,
          idx_v0, rows_v0, idx_v1, rows_v1, sem0, sem1):
    wid = lax.axis_index("s") * NC + lax.axis_index("c")
    s1_base = (wid // W_C) * S1_PER_W
    c_base = (wid % W_C) * CG_PER_W

    def coords(t):
        return s1_base + t // 2, c_base + lax.rem(t, 2) * UNITS_PER_BATCH

    def fetch_fire(t, idx_b, rows_b, sem):
        s1, c0 = coords(t)
        pltpu.sync_copy(idx_hbm.at[s1, pl.ds(c0 * IDX_W, BATCH_ROWS)], idx_b)
        for j in range(UNITS_PER_BATCH):
            pltpu.async_copy(
                table_hbm.at[idx_b.at[pl.ds(j * IDX_W, IDX_W)]],
                rows_b.at[pl.ds(j * IDX_W, IDX_W)],
                sem,
            )

    def process(t, idx_b, rows_b, sem):
        s1, c0 = coords(t)
        pltpu.make_async_copy(table_hbm.at[idx_b], rows_b, sem).wait()

        @pl.loop(0, BATCH_ROWS)
        def _row(r):
            for k in range(MODEL_DIM // LANE):
                sl = pl.ds(k * LANE, LANE)
                rows_b[r, sl] = rows_b[r, sl] * SCALE

        for j in range(UNITS_PER_BATCH):
            pltpu.sync_copy(
                rows_b.at[pl.ds(j * IDX_W, IDX_W)],
                out_hbm.at[pl.ds((c0 + j) * IDX_W, IDX_W), s1],
            )

    fetch_fire(0, idx_v0, rows_v0, sem0)

    @pl.loop(0, NBATCH // 2)
    def _step(i):
        fetch_fire(2 * i + 1, idx_v1, rows_v1, sem1)
        process(2 * i, idx_v0, rows_v0, sem0)

        @pl.when(i < NBATCH // 2 - 1)
        def _prefetch():
            fetch_fire(2 * i + 2, idx_v0, rows_v0, sem0)

        process(2 * i + 1, idx_v1, rows_v1, sem1)


def kernel(x, table):
    xt = x.T.astype(jnp.int32)

    run = pl.kernel(
        _body,
        out_type=jax.ShapeDtypeStruct((S0, S1, MODEL_DIM), jnp.float32),
        mesh=plsc.VectorSubcoreMesh(core_axis_name="c", subcore_axis_name="s"),
        scratch_types=[
            pltpu.VMEM((BATCH_ROWS,), jnp.int32),
            pltpu.VMEM((BATCH_ROWS, MODEL_DIM), jnp.float32),
            pltpu.VMEM((BATCH_ROWS,), jnp.int32),
            pltpu.VMEM((BATCH_ROWS, MODEL_DIM), jnp.float32),
            pltpu.SemaphoreType.DMA,
            pltpu.SemaphoreType.DMA,
        ],
        compiler_params=pltpu.CompilerParams(use_tc_tiling_on_sc=False),
    )
    return run(xt, table)


# speedup vs baseline: 1.8689x; 1.0253x over previous
import jax
import jax.numpy as jnp
from jax import lax
from jax.experimental import pallas as pl
from jax.experimental.pallas import tpu as pltpu
from jax.experimental.pallas import tpu_sc as plsc

MODEL_DIM = 64
SCALE = 8.0

NC = 2
NS = 16
LANE = 16
IDX_W = 128
UNITS_PER_BATCH = 4
BATCH_ROWS = UNITS_PER_BATCH * IDX_W

S0 = 4096
S1 = 200
CBLK = S0 // IDX_W
W_S1 = 8
W_C = 4
S1_PER_W = S1 // W_S1
CG_PER_W = CBLK // W_C
NBATCH = S1_PER_W * (CG_PER_W // UNITS_PER_BATCH)


def _body(idx_hbm, table_hbm, out_hbm,
          idx_all, rows_v0, rows_v1, sem0, sem1):
    wid = lax.axis_index("s") * NC + lax.axis_index("c")
    s1_base = (wid // W_C) * S1_PER_W
    c_base = (wid % W_C) * CG_PER_W

    pltpu.sync_copy(
        idx_hbm.at[pl.ds(s1_base, S1_PER_W),
                   pl.ds(c_base * IDX_W, CG_PER_W * IDX_W)],
        idx_all)

    def coords(t):
        return s1_base + t // 2, c_base + lax.rem(t, 2) * UNITS_PER_BATCH

    def idx_slice(t, j):
        return idx_all.at[t // 2,
                          pl.ds(lax.rem(t, 2) * BATCH_ROWS + j * IDX_W, IDX_W)]

    def fetch_fire(t, rows_b, sem):
        for j in range(UNITS_PER_BATCH):
            pltpu.async_copy(
                table_hbm.at[idx_slice(t, j)],
                rows_b.at[pl.ds(j * IDX_W, IDX_W)],
                sem,
            )

    def process(t, rows_b, sem):
        s1, c0 = coords(t)
        pltpu.make_async_copy(
            table_hbm.at[idx_all.at[0, pl.ds(0, BATCH_ROWS)]],
            rows_b, sem).wait()

        @pl.loop(0, BATCH_ROWS)
        def _row(r):
            for k in range(MODEL_DIM // LANE):
                sl = pl.ds(k * LANE, LANE)
                rows_b[r, sl] = rows_b[r, sl] * SCALE

        for j in range(UNITS_PER_BATCH):
            pltpu.sync_copy(
                rows_b.at[pl.ds(j * IDX_W, IDX_W)],
                out_hbm.at[pl.ds((c0 + j) * IDX_W, IDX_W), s1],
            )

    fetch_fire(0, rows_v0, sem0)

    @pl.loop(0, NBATCH // 2)
    def _step(i):
        fetch_fire(2 * i + 1, rows_v1, sem1)
        process(2 * i, rows_v0, sem0)

        @pl.when(i < NBATCH // 2 - 1)
        def _prefetch():
            fetch_fire(2 * i + 2, rows_v0, sem0)

        process(2 * i + 1, rows_v1, sem1)


def kernel(x, table):
    xt = x.T.astype(jnp.int32)

    run = pl.kernel(
        _body,
        out_type=jax.ShapeDtypeStruct((S0, S1, MODEL_DIM), jnp.float32),
        mesh=plsc.VectorSubcoreMesh(core_axis_name="c", subcore_axis_name="s"),
        scratch_types=[
            pltpu.VMEM((S1_PER_W, CG_PER_W * IDX_W), jnp.int32),
            pltpu.VMEM((BATCH_ROWS, MODEL_DIM), jnp.float32),
            pltpu.VMEM((BATCH_ROWS, MODEL_DIM), jnp.float32),
            pltpu.SemaphoreType.DMA,
            pltpu.SemaphoreType.DMA,
        ],
        compiler_params=pltpu.CompilerParams(use_tc_tiling_on_sc=False),
    )
    return run(xt, table)
